# Initial kernel scaffold; baseline (speedup 1.0000x reference)
import jax
import jax.numpy as jnp
from jax import lax
from jax.experimental import pallas as pl
from jax.experimental.pallas import tpu as pltpu, tpu_sc as plsc

N = 10000
NP = 10240
E = 320000
EPAD = 323584
D = 128
DH = 64
OUT = 64

NC, NS = 2, 16
ROWS_PER_TILE = NP // NS
CH = 128
C_SCAT = EPAD // (NS * CH)
C_DEG = EPAD // (NC * NS * CH)

_mesh = plsc.VectorSubcoreMesh(
    core_axis_name="c", subcore_axis_name="s", num_cores=NC, num_subcores=NS)


def _degree_body(dst_hbm, degp_hbm, deg_sh, dst_v, ones_v, zero_v):
    c = lax.axis_index("c")
    s = lax.axis_index("s")
    w = c * NS + s
    r0 = s * ROWS_PER_TILE

    def _fill(i, carry):
        ones_v[pl.ds(i * 16, 16)] = jnp.ones((16,), jnp.float32)
        return carry

    lax.fori_loop(0, CH // 16, _fill, 0)

    def _zfill(i, carry):
        zero_v[pl.ds(i * 16, 16)] = jnp.zeros((16,), jnp.float32)
        return carry

    lax.fori_loop(0, ROWS_PER_TILE // 16, _zfill, 0)
    pltpu.sync_copy(zero_v, deg_sh.at[pl.ds(r0, ROWS_PER_TILE)])
    pltpu.sync_copy(dst_hbm.at[w], dst_v)
    plsc.subcore_barrier()

    def _scat(j, carry):
        pltpu.sync_copy(ones_v, deg_sh.at[dst_v.at[j]], add=True)
        return carry

    lax.fori_loop(0, C_DEG, _scat, 0)
    plsc.subcore_barrier()
    pltpu.sync_copy(deg_sh.at[pl.ds(r0, ROWS_PER_TILE)],
                    degp_hbm.at[c, pl.ds(r0, ROWS_PER_TILE)])


_degree_call = pl.kernel(
    _degree_body,
    out_type=jax.ShapeDtypeStruct((NC, NP), jnp.float32),
    mesh=_mesh,
    scratch_types=[
        pltpu.VMEM_SHARED((NP,), jnp.float32),
        pltpu.VMEM((C_DEG, CH), jnp.int32),
        pltpu.VMEM((CH,), jnp.float32),
        pltpu.VMEM((ROWS_PER_TILE,), jnp.float32),
    ],
)


def _scatter_body(z_hbm, src_hbm, dst_hbm, out_hbm, z_sh, acc_sh,
                  src_v, dst_v, rows_v):
    c = lax.axis_index("c")
    s = lax.axis_index("s")
    r0 = s * ROWS_PER_TILE
    pltpu.sync_copy(z_hbm.at[c, pl.ds(r0, ROWS_PER_TILE)],
                    z_sh.at[pl.ds(r0, ROWS_PER_TILE)])
    pltpu.sync_copy(z_hbm.at[c, pl.ds(r0, ROWS_PER_TILE)],
                    acc_sh.at[pl.ds(r0, ROWS_PER_TILE)])
    pltpu.sync_copy(src_hbm.at[s], src_v)
    pltpu.sync_copy(dst_hbm.at[s], dst_v)
    plsc.subcore_barrier()

    def _edge_chunk(j, carry):
        pltpu.sync_copy(z_sh.at[src_v.at[j]], rows_v)
        pltpu.sync_copy(rows_v, acc_sh.at[dst_v.at[j]], add=True)
        return carry

    lax.fori_loop(0, C_SCAT, _edge_chunk, 0)
    plsc.subcore_barrier()
    pltpu.sync_copy(acc_sh.at[pl.ds(r0, ROWS_PER_TILE)],
                    out_hbm.at[c, pl.ds(r0, ROWS_PER_TILE)])


_scatter_call = pl.kernel(
    _scatter_body,
    out_type=jax.ShapeDtypeStruct((NC, NP, DH), jnp.float32),
    mesh=_mesh,
    scratch_types=[
        pltpu.VMEM_SHARED((NP, DH), jnp.float32),
        pltpu.VMEM_SHARED((NP, DH), jnp.float32),
        pltpu.VMEM((C_SCAT, CH), jnp.int32),
        pltpu.VMEM((C_SCAT, CH), jnp.int32),
        pltpu.VMEM((CH, DH), jnp.float32),
    ],
)


R = 1280
GRID = NP // R


def _stage_a(x_ref, wf_ref, bf_ref, wg1_ref, degp_ref, z_ref, dinv_ref):
    deg = degp_ref[0, :] + degp_ref[1, :] + 1.0
    dinv = lax.rsqrt(deg).reshape(R, 1)
    h = jnp.maximum(
        jnp.dot(x_ref[...], wf_ref[...], preferred_element_type=jnp.float32)
        + bf_ref[...], 0.0)
    z = jnp.dot(h, wg1_ref[...], preferred_element_type=jnp.float32) * dinv
    z_ref[0] = z[:, :DH]
    z_ref[1] = z[:, DH:]
    dinv_ref[...] = dinv


_stage_a_call = pl.pallas_call(
    _stage_a,
    grid=(GRID,),
    in_specs=[
        pl.BlockSpec((R, D), lambda i: (i, 0)),
        pl.BlockSpec((D, D), lambda i: (0, 0)),
        pl.BlockSpec((1, D), lambda i: (0, 0)),
        pl.BlockSpec((D, D), lambda i: (0, 0)),
        pl.BlockSpec((NC, R), lambda i: (0, i)),
    ],
    out_specs=[
        pl.BlockSpec((NC, R, DH), lambda i: (0, i, 0)),
        pl.BlockSpec((R, 1), lambda i: (i, 0)),
    ],
    out_shape=[
        jax.ShapeDtypeStruct((NC, NP, DH), jnp.float32),
        jax.ShapeDtypeStruct((NP, 1), jnp.float32),
    ],
)


def _stage_b(acc_ref, dinv_ref, bg1_ref, wg2_ref, z2_ref):
    dinv = dinv_ref[...]
    t = jnp.concatenate([acc_ref[0], acc_ref[1]], axis=1) * dinv + bg1_ref[...]
    h1 = jnp.maximum(t, 0.0)
    z2 = jnp.dot(h1, wg2_ref[...], preferred_element_type=jnp.float32) * dinv
    z2_ref[0] = z2[:, :DH]
    z2_ref[1] = z2[:, DH:]


_stage_b_call = pl.pallas_call(
    _stage_b,
    grid=(GRID,),
    in_specs=[
        pl.BlockSpec((NC, R, DH), lambda i: (0, i, 0)),
        pl.BlockSpec((R, 1), lambda i: (i, 0)),
        pl.BlockSpec((1, D), lambda i: (0, 0)),
        pl.BlockSpec((D, D), lambda i: (0, 0)),
    ],
    out_specs=pl.BlockSpec((NC, R, DH), lambda i: (0, i, 0)),
    out_shape=jax.ShapeDtypeStruct((NC, NP, DH), jnp.float32),
)


def _stage_c(acc_ref, dinv_ref, bg2_ref, wfin_ref, bfin_ref, out_ref):
    dinv = dinv_ref[...]
    t = jnp.concatenate([acc_ref[0], acc_ref[1]], axis=1) * dinv + bg2_ref[...]
    h2 = jnp.maximum(t, 0.0)
    f = (jnp.dot(h2, wfin_ref[...], preferred_element_type=jnp.float32)
         + bfin_ref[...])
    m = jnp.max(f, axis=1, keepdims=True)
    lse = jnp.log(jnp.sum(jnp.exp(f - m), axis=1, keepdims=True))
    out_ref[...] = f - m - lse


_stage_c_call = pl.pallas_call(
    _stage_c,
    grid=(GRID,),
    in_specs=[
        pl.BlockSpec((NC, R, DH), lambda i: (0, i, 0)),
        pl.BlockSpec((R, 1), lambda i: (i, 0)),
        pl.BlockSpec((1, D), lambda i: (0, 0)),
        pl.BlockSpec((D, OUT), lambda i: (0, 0)),
        pl.BlockSpec((1, OUT), lambda i: (0, 0)),
    ],
    out_specs=pl.BlockSpec((R, OUT), lambda i: (i, 0)),
    out_shape=jax.ShapeDtypeStruct((NP, OUT), jnp.float32),
)


@jax.jit
def kernel(x, edge_index, W_first, b_first, W_gc1, b_gc1, W_gc2, b_gc2,
           W_final, b_final):
    src = edge_index[0].astype(jnp.int32)
    dst = edge_index[1].astype(jnp.int32)
    npad = EPAD - E
    src_p = jnp.concatenate([src, jnp.zeros((npad,), jnp.int32)])
    dst_p = jnp.concatenate([dst, jnp.full((npad,), N, jnp.int32)])
    src_s = src_p.reshape(NS, C_SCAT, CH)
    dst_s = dst_p.reshape(NS, C_SCAT, CH)
    dst_d = dst_p.reshape(NC * NS, C_DEG, CH)
    x_p = jnp.pad(x, ((0, NP - N), (0, 0)))

    degp = _degree_call(dst_d)
    z, dinv = _stage_a_call(x_p, W_first, b_first.reshape(1, D), W_gc1, degp)
    acc1 = _scatter_call(z, src_s, dst_s)
    z2 = _stage_b_call(acc1, dinv, b_gc1.reshape(1, D), W_gc2)
    acc2 = _scatter_call(z2, src_s, dst_s)
    out = _stage_c_call(acc2, dinv, b_gc2.reshape(1, D), W_final,
                        b_final.reshape(1, OUT))
    return out[:N]


# speedup vs baseline: 8.5355x; 8.5355x over previous
import jax
import jax.numpy as jnp
from jax import lax
from jax.experimental import pallas as pl
from jax.experimental.pallas import tpu as pltpu, tpu_sc as plsc

N = 10000
NP = 10240
E = 320000
EPAD = 327680
D = 128
OUT = 64

NC, NS = 2, 16
NW = NC * NS
ROWS_PER_TILE = NP // NS
CH = 128
G = 16
C_W = EPAD // (NW * CH)
NG = C_W // G

_mesh = plsc.VectorSubcoreMesh(
    core_axis_name="c", subcore_axis_name="s", num_cores=NC, num_subcores=NS)


def _degree_body(dst_hbm, degp_hbm, deg_sh, dst_v, ones_v, zero_v):
    c = lax.axis_index("c")
    s = lax.axis_index("s")
    w = c * NS + s
    r0 = s * ROWS_PER_TILE

    def _fill(i, carry):
        ones_v[pl.ds(i * 16, 16)] = jnp.ones((16,), jnp.float32)
        return carry

    lax.fori_loop(0, CH // 16, _fill, 0)

    def _zfill(i, carry):
        zero_v[pl.ds(i * 16, 16)] = jnp.zeros((16,), jnp.float32)
        return carry

    lax.fori_loop(0, ROWS_PER_TILE // 16, _zfill, 0)
    pltpu.sync_copy(zero_v, deg_sh.at[pl.ds(r0, ROWS_PER_TILE)])
    plsc.subcore_barrier()

    def _group(g, carry):
        pltpu.sync_copy(dst_hbm.at[w, pl.ds(g * G, G)], dst_v)

        def _scat(j, carry2):
            pltpu.sync_copy(ones_v, deg_sh.at[dst_v.at[j]], add=True)
            return carry2

        lax.fori_loop(0, G, _scat, 0)
        return carry

    lax.fori_loop(0, NG, _group, 0)
    plsc.subcore_barrier()
    pltpu.sync_copy(deg_sh.at[pl.ds(r0, ROWS_PER_TILE)],
                    degp_hbm.at[c, pl.ds(r0, ROWS_PER_TILE)])


_degree_call = pl.kernel(
    _degree_body,
    out_type=jax.ShapeDtypeStruct((NC, NP), jnp.float32),
    mesh=_mesh,
    scratch_types=[
        pltpu.VMEM_SHARED((NP,), jnp.float32),
        pltpu.VMEM((G, CH), jnp.int32),
        pltpu.VMEM((CH,), jnp.float32),
        pltpu.VMEM((ROWS_PER_TILE,), jnp.float32),
    ],
)


def _scatter_body(z_hbm, src_hbm, dst_hbm, out_hbm, acc_sh,
                  src_v, dst_v, rows_v):
    c = lax.axis_index("c")
    s = lax.axis_index("s")
    w = c * NS + s
    r0 = s * ROWS_PER_TILE

    def _zrow(r, carry):
        def _zcol(k, carry2):
            rows_v[r, pl.ds(k * 16, 16)] = jnp.zeros((16,), jnp.float32)
            return carry2

        return lax.fori_loop(0, D // 16, _zcol, carry)

    lax.fori_loop(0, CH, _zrow, 0)

    def _zcopy(t, carry):
        pltpu.sync_copy(rows_v, acc_sh.at[pl.ds(r0 + t * CH, CH)])
        return carry

    lax.fori_loop(0, ROWS_PER_TILE // CH, _zcopy, 0)
    plsc.subcore_barrier()

    def _group(g, carry):
        pltpu.sync_copy(src_hbm.at[w, pl.ds(g * G, G)], src_v)
        pltpu.sync_copy(dst_hbm.at[w, pl.ds(g * G, G)], dst_v)

        def _edge_chunk(j, carry2):
            pltpu.sync_copy(z_hbm.at[src_v.at[j]], rows_v)
            pltpu.sync_copy(rows_v, acc_sh.at[dst_v.at[j]], add=True)
            return carry2

        lax.fori_loop(0, G, _edge_chunk, 0)
        return carry

    lax.fori_loop(0, NG, _group, 0)
    plsc.subcore_barrier()
    pltpu.sync_copy(acc_sh.at[pl.ds(r0, ROWS_PER_TILE)],
                    out_hbm.at[c, pl.ds(r0, ROWS_PER_TILE)])


_scatter_call = pl.kernel(
    _scatter_body,
    out_type=jax.ShapeDtypeStruct((NC, NP, D), jnp.float32),
    mesh=_mesh,
    scratch_types=[
        pltpu.VMEM_SHARED((NP, D), jnp.float32),
        pltpu.VMEM((G, CH), jnp.int32),
        pltpu.VMEM((G, CH), jnp.int32),
        pltpu.VMEM((CH, D), jnp.float32),
    ],
)


R = 1280
GRID = NP // R


def _stage_a(x_ref, wf_ref, bf_ref, wg1_ref, degp_ref, z_ref, dinv_ref):
    deg = degp_ref[0, :] + degp_ref[1, :] + 1.0
    dinv = lax.rsqrt(deg).reshape(R, 1)
    h = jnp.maximum(
        jnp.dot(x_ref[...], wf_ref[...], preferred_element_type=jnp.float32)
        + bf_ref[...], 0.0)
    z_ref[...] = jnp.dot(h, wg1_ref[...],
                         preferred_element_type=jnp.float32) * dinv
    dinv_ref[...] = dinv


_stage_a_call = pl.pallas_call(
    _stage_a,
    grid=(GRID,),
    in_specs=[
        pl.BlockSpec((R, D), lambda i: (i, 0)),
        pl.BlockSpec((D, D), lambda i: (0, 0)),
        pl.BlockSpec((1, D), lambda i: (0, 0)),
        pl.BlockSpec((D, D), lambda i: (0, 0)),
        pl.BlockSpec((NC, R), lambda i: (0, i)),
    ],
    out_specs=[
        pl.BlockSpec((R, D), lambda i: (i, 0)),
        pl.BlockSpec((R, 1), lambda i: (i, 0)),
    ],
    out_shape=[
        jax.ShapeDtypeStruct((NP, D), jnp.float32),
        jax.ShapeDtypeStruct((NP, 1), jnp.float32),
    ],
)


def _stage_b(p_ref, z_ref, dinv_ref, bg1_ref, wg2_ref, z2_ref):
    dinv = dinv_ref[...]
    t = (p_ref[0] + p_ref[1] + z_ref[...]) * dinv + bg1_ref[...]
    h1 = jnp.maximum(t, 0.0)
    z2_ref[...] = jnp.dot(h1, wg2_ref[...],
                          preferred_element_type=jnp.float32) * dinv


_stage_b_call = pl.pallas_call(
    _stage_b,
    grid=(GRID,),
    in_specs=[
        pl.BlockSpec((NC, R, D), lambda i: (0, i, 0)),
        pl.BlockSpec((R, D), lambda i: (i, 0)),
        pl.BlockSpec((R, 1), lambda i: (i, 0)),
        pl.BlockSpec((1, D), lambda i: (0, 0)),
        pl.BlockSpec((D, D), lambda i: (0, 0)),
    ],
    out_specs=pl.BlockSpec((R, D), lambda i: (i, 0)),
    out_shape=jax.ShapeDtypeStruct((NP, D), jnp.float32),
)


def _stage_c(p_ref, z2_ref, dinv_ref, bg2_ref, wfin_ref, bfin_ref, out_ref):
    dinv = dinv_ref[...]
    t = (p_ref[0] + p_ref[1] + z2_ref[...]) * dinv + bg2_ref[...]
    h2 = jnp.maximum(t, 0.0)
    f = (jnp.dot(h2, wfin_ref[...], preferred_element_type=jnp.float32)
         + bfin_ref[...])
    m = jnp.max(f, axis=1, keepdims=True)
    lse = jnp.log(jnp.sum(jnp.exp(f - m), axis=1, keepdims=True))
    out_ref[...] = f - m - lse


_stage_c_call = pl.pallas_call(
    _stage_c,
    grid=(GRID,),
    in_specs=[
        pl.BlockSpec((NC, R, D), lambda i: (0, i, 0)),
        pl.BlockSpec((R, D), lambda i: (i, 0)),
        pl.BlockSpec((R, 1), lambda i: (i, 0)),
        pl.BlockSpec((1, D), lambda i: (0, 0)),
        pl.BlockSpec((D, OUT), lambda i: (0, 0)),
        pl.BlockSpec((1, OUT), lambda i: (0, 0)),
    ],
    out_specs=pl.BlockSpec((R, OUT), lambda i: (i, 0)),
    out_shape=jax.ShapeDtypeStruct((NP, OUT), jnp.float32),
)


@jax.jit
def kernel(x, edge_index, W_first, b_first, W_gc1, b_gc1, W_gc2, b_gc2,
           W_final, b_final):
    src = edge_index[0].astype(jnp.int32)
    dst = edge_index[1].astype(jnp.int32)
    npad = EPAD - E
    src_p = jnp.concatenate([src, jnp.zeros((npad,), jnp.int32)])
    dst_p = jnp.concatenate([dst, jnp.full((npad,), N, jnp.int32)])
    src_w = src_p.reshape(NW, C_W, CH)
    dst_w = dst_p.reshape(NW, C_W, CH)
    x_p = jnp.pad(x, ((0, NP - N), (0, 0)))

    degp = _degree_call(dst_w)
    z, dinv = _stage_a_call(x_p, W_first, b_first.reshape(1, D), W_gc1, degp)
    p1 = _scatter_call(z, src_w, dst_w)
    z2 = _stage_b_call(p1, z, dinv, b_gc1.reshape(1, D), W_gc2)
    p2 = _scatter_call(z2, src_w, dst_w)
    out = _stage_c_call(p2, z2, dinv, b_gc2.reshape(1, D), W_final,
                        b_final.reshape(1, OUT))
    return out[:N]


# speedup vs baseline: 9.1440x; 1.0713x over previous
import jax
import jax.numpy as jnp
from jax import lax
from jax.experimental import pallas as pl
from jax.experimental.pallas import tpu as pltpu, tpu_sc as plsc

N = 10000
NP = 10240
E = 320000
EPAD = 327680
D = 128
OUT = 64

NC, NS = 2, 16
NW = NC * NS
ROWS_PER_TILE = NP // NS
CH = 128
G = 16
C_W = EPAD // (NW * CH)
NG = C_W // G

_mesh = plsc.VectorSubcoreMesh(
    core_axis_name="c", subcore_axis_name="s", num_cores=NC, num_subcores=NS)


def _degree_body(dst_hbm, degp_hbm, deg_sh, dst_v, ones_v, zero_v):
    c = lax.axis_index("c")
    s = lax.axis_index("s")
    w = c * NS + s
    r0 = s * ROWS_PER_TILE

    def _fill(i, carry):
        ones_v[pl.ds(i * 16, 16)] = jnp.ones((16,), jnp.float32)
        return carry

    lax.fori_loop(0, CH // 16, _fill, 0)

    def _zfill(i, carry):
        zero_v[pl.ds(i * 16, 16)] = jnp.zeros((16,), jnp.float32)
        return carry

    lax.fori_loop(0, ROWS_PER_TILE // 16, _zfill, 0)
    pltpu.sync_copy(zero_v, deg_sh.at[pl.ds(r0, ROWS_PER_TILE)])
    plsc.subcore_barrier()

    def _group(g, carry):
        pltpu.sync_copy(dst_hbm.at[w, pl.ds(g * G, G)], dst_v)

        def _scat(j, carry2):
            pltpu.sync_copy(ones_v, deg_sh.at[dst_v.at[j]], add=True)
            return carry2

        lax.fori_loop(0, G, _scat, 0)
        return carry

    lax.fori_loop(0, NG, _group, 0)
    plsc.subcore_barrier()
    pltpu.sync_copy(deg_sh.at[pl.ds(r0, ROWS_PER_TILE)],
                    degp_hbm.at[c, pl.ds(r0, ROWS_PER_TILE)])


_degree_call = pl.kernel(
    _degree_body,
    out_type=jax.ShapeDtypeStruct((NC, NP), jnp.float32),
    mesh=_mesh,
    scratch_types=[
        pltpu.VMEM_SHARED((NP,), jnp.float32),
        pltpu.VMEM((G, CH), jnp.int32),
        pltpu.VMEM((CH,), jnp.float32),
        pltpu.VMEM((ROWS_PER_TILE,), jnp.float32),
    ],
)


def _scatter_body(z_hbm, src_hbm, dst_hbm, out_hbm, acc_sh,
                  src_v, dst_v, rows_v, sem_g, sem_s):
    c = lax.axis_index("c")
    s = lax.axis_index("s")
    w = c * NS + s
    r0 = s * ROWS_PER_TILE

    def _zrow(r, carry):
        def _zcol(k, carry2):
            rows_v[0, r, pl.ds(k * 16, 16)] = jnp.zeros((16,), jnp.float32)
            return carry2

        return lax.fori_loop(0, D // 16, _zcol, carry)

    lax.fori_loop(0, CH, _zrow, 0)

    def _zcopy(t, carry):
        pltpu.sync_copy(rows_v.at[0], acc_sh.at[pl.ds(r0 + t * CH, CH)])
        return carry

    lax.fori_loop(0, ROWS_PER_TILE // CH, _zcopy, 0)
    plsc.subcore_barrier()

    def _group(g, carry):
        pltpu.sync_copy(src_hbm.at[w, pl.ds(g * G, G)], src_v)
        pltpu.sync_copy(dst_hbm.at[w, pl.ds(g * G, G)], dst_v)

        gath = [None, None]
        scat = [None, None]
        gath[0] = pltpu.async_copy(z_hbm.at[src_v.at[0]], rows_v.at[0], sem_g)
        for j in range(G):
            b = j % 2
            gath[b].wait()
            if j >= 1:
                scat[1 - b].wait()
            scat[b] = pltpu.async_copy(
                rows_v.at[b], acc_sh.at[dst_v.at[j]], sem_s, add=True)
            if j + 1 < G:
                gath[1 - b] = pltpu.async_copy(
                    z_hbm.at[src_v.at[j + 1]], rows_v.at[1 - b], sem_g)
        scat[(G - 1) % 2].wait()
        return carry

    lax.fori_loop(0, NG, _group, 0)
    plsc.subcore_barrier()
    pltpu.sync_copy(acc_sh.at[pl.ds(r0, ROWS_PER_TILE)],
                    out_hbm.at[c, pl.ds(r0, ROWS_PER_TILE)])


_scatter_call = pl.kernel(
    _scatter_body,
    out_type=jax.ShapeDtypeStruct((NC, NP, D), jnp.float32),
    mesh=_mesh,
    scratch_types=[
        pltpu.VMEM_SHARED((NP, D), jnp.float32),
        pltpu.VMEM((G, CH), jnp.int32),
        pltpu.VMEM((G, CH), jnp.int32),
        pltpu.VMEM((2, CH, D), jnp.float32),
        pltpu.SemaphoreType.DMA,
        pltpu.SemaphoreType.DMA,
    ],
)


R = 1280
GRID = NP // R


def _stage_a(x_ref, wf_ref, bf_ref, wg1_ref, degp_ref, z_ref, dinv_ref):
    deg = degp_ref[0, :] + degp_ref[1, :] + 1.0
    dinv = lax.rsqrt(deg).reshape(R, 1)
    h = jnp.maximum(
        jnp.dot(x_ref[...], wf_ref[...], preferred_element_type=jnp.float32)
        + bf_ref[...], 0.0)
    z_ref[...] = jnp.dot(h, wg1_ref[...],
                         preferred_element_type=jnp.float32) * dinv
    dinv_ref[...] = dinv


_stage_a_call = pl.pallas_call(
    _stage_a,
    grid=(GRID,),
    in_specs=[
        pl.BlockSpec((R, D), lambda i: (i, 0)),
        pl.BlockSpec((D, D), lambda i: (0, 0)),
        pl.BlockSpec((1, D), lambda i: (0, 0)),
        pl.BlockSpec((D, D), lambda i: (0, 0)),
        pl.BlockSpec((NC, R), lambda i: (0, i)),
    ],
    out_specs=[
        pl.BlockSpec((R, D), lambda i: (i, 0)),
        pl.BlockSpec((R, 1), lambda i: (i, 0)),
    ],
    out_shape=[
        jax.ShapeDtypeStruct((NP, D), jnp.float32),
        jax.ShapeDtypeStruct((NP, 1), jnp.float32),
    ],
)


def _stage_b(p_ref, z_ref, dinv_ref, bg1_ref, wg2_ref, z2_ref):
    dinv = dinv_ref[...]
    t = (p_ref[0] + p_ref[1] + z_ref[...]) * dinv + bg1_ref[...]
    h1 = jnp.maximum(t, 0.0)
    z2_ref[...] = jnp.dot(h1, wg2_ref[...],
                          preferred_element_type=jnp.float32) * dinv


_stage_b_call = pl.pallas_call(
    _stage_b,
    grid=(GRID,),
    in_specs=[
        pl.BlockSpec((NC, R, D), lambda i: (0, i, 0)),
        pl.BlockSpec((R, D), lambda i: (i, 0)),
        pl.BlockSpec((R, 1), lambda i: (i, 0)),
        pl.BlockSpec((1, D), lambda i: (0, 0)),
        pl.BlockSpec((D, D), lambda i: (0, 0)),
    ],
    out_specs=pl.BlockSpec((R, D), lambda i: (i, 0)),
    out_shape=jax.ShapeDtypeStruct((NP, D), jnp.float32),
)


def _stage_c(p_ref, z2_ref, dinv_ref, bg2_ref, wfin_ref, bfin_ref, out_ref):
    dinv = dinv_ref[...]
    t = (p_ref[0] + p_ref[1] + z2_ref[...]) * dinv + bg2_ref[...]
    h2 = jnp.maximum(t, 0.0)
    f = (jnp.dot(h2, wfin_ref[...], preferred_element_type=jnp.float32)
         + bfin_ref[...])
    m = jnp.max(f, axis=1, keepdims=True)
    lse = jnp.log(jnp.sum(jnp.exp(f - m), axis=1, keepdims=True))
    out_ref[...] = f - m - lse


_stage_c_call = pl.pallas_call(
    _stage_c,
    grid=(GRID,),
    in_specs=[
        pl.BlockSpec((NC, R, D), lambda i: (0, i, 0)),
        pl.BlockSpec((R, D), lambda i: (i, 0)),
        pl.BlockSpec((R, 1), lambda i: (i, 0)),
        pl.BlockSpec((1, D), lambda i: (0, 0)),
        pl.BlockSpec((D, OUT), lambda i: (0, 0)),
        pl.BlockSpec((1, OUT), lambda i: (0, 0)),
    ],
    out_specs=pl.BlockSpec((R, OUT), lambda i: (i, 0)),
    out_shape=jax.ShapeDtypeStruct((NP, OUT), jnp.float32),
)


@jax.jit
def kernel(x, edge_index, W_first, b_first, W_gc1, b_gc1, W_gc2, b_gc2,
           W_final, b_final):
    src = edge_index[0].astype(jnp.int32)
    dst = edge_index[1].astype(jnp.int32)
    npad = EPAD - E
    src_p = jnp.concatenate([src, jnp.zeros((npad,), jnp.int32)])
    dst_p = jnp.concatenate([dst, jnp.full((npad,), N, jnp.int32)])
    src_w = src_p.reshape(NW, C_W, CH)
    dst_w = dst_p.reshape(NW, C_W, CH)
    x_p = jnp.pad(x, ((0, NP - N), (0, 0)))

    degp = _degree_call(dst_w)
    z, dinv = _stage_a_call(x_p, W_first, b_first.reshape(1, D), W_gc1, degp)
    p1 = _scatter_call(z, src_w, dst_w)
    z2 = _stage_b_call(p1, z, dinv, b_gc1.reshape(1, D), W_gc2)
    p2 = _scatter_call(z2, src_w, dst_w)
    out = _stage_c_call(p2, z2, dinv, b_gc2.reshape(1, D), W_final,
                        b_final.reshape(1, OUT))
    return out[:N]
